# baseline (speedup 1.0000x reference)
import jax
import jax.numpy as jnp
from jax.experimental import pallas as pl
from jax.experimental.pallas import tpu as pltpu

_TOP_K = 2
_LAMBDA_RANKING = 0.3
_MARGIN = 0.1
_W_TOP10 = 3.0
_W_TOP11_30 = 1.5
_W_OTHERS = 0.5
_TOP_N = 10
_LANES = 512


def _loss_kernel(p_ref, g_ref, out_ref, *, n_tokens, n_experts, grid):
    i = pl.program_id(0)

    @pl.when(i == 0)
    def _init():
        out_ref[0] = 0.0
        out_ref[1] = 0.0
        out_ref[2] = 0.0
        out_ref[3] = 0.0

    p = p_ref[...].T
    g = g_ref[...].T
    E = n_experts
    L = p.shape[1]

    b = jax.lax.bitcast_convert_type(g, jnp.int32)
    k = jnp.where(b >= 0, b, b ^ jnp.int32(0x7FFFFFFF))
    aidx = jax.lax.broadcasted_iota(jnp.int32, (E, L), 0)
    c = (k & jnp.int32(~63)) | (jnp.int32(E - 1) - aidx)

    ranks = jnp.zeros((E, L), jnp.int32)
    for bb in range(E):
        cb = c[bb : bb + 1, :]
        ranks = ranks + (cb > c).astype(jnp.int32)

    lt2 = ranks < _TOP_K
    lt10 = ranks < _TOP_N
    lt30 = ranks < 30
    base = jnp.maximum(p, 0.0) + jnp.log(1.0 + jnp.exp(-jnp.abs(p)))
    w = _W_OTHERS + jnp.where(lt30, 1.0, 0.0) + jnp.where(lt10, 1.5, 0.0)
    wsum = w * base - _W_TOP10 * jnp.where(lt2, p, 0.0)
    s_wbce = jnp.sum(wsum)

    qs = []
    vs = []
    for r in range(_TOP_N):
        m = ranks == r
        qs.append(jnp.sum(jnp.where(m, p, 0.0), axis=0, keepdims=True))
        vs.append(jnp.sum(jnp.where(m, g, 0.0), axis=0, keepdims=True))

    pair = jnp.zeros((1, L), jnp.float32)
    cnt = jnp.zeros((1, L), jnp.float32)
    for a in range(_TOP_N):
        for b2 in range(a + 1, _TOP_N):
            valid = vs[a] > vs[b2]
            t = jnp.maximum(_MARGIN - (qs[a] - qs[b2]), 0.0)
            pair = pair + jnp.where(valid, t, 0.0)
            cnt = cnt + jnp.where(valid, 1.0, 0.0)

    out_ref[0] = out_ref[0] + s_wbce
    out_ref[1] = out_ref[1] + jnp.sum(pair)
    out_ref[2] = out_ref[2] + jnp.sum(cnt)

    @pl.when(i == grid - 1)
    def _final():
        wb = out_ref[0] * (1.0 / float(n_tokens * n_experts))
        nv = out_ref[2]
        rank_loss = jnp.where(nv > 0.0, out_ref[1] / jnp.maximum(nv, 1.0), 0.0)
        out_ref[3] = wb + _LAMBDA_RANKING * rank_loss


def kernel(predictions, gate_logits):
    N, E = predictions.shape
    L = min(_LANES, N)
    grid = N // L

    import functools

    out = pl.pallas_call(
        functools.partial(_loss_kernel, n_tokens=N, n_experts=E, grid=grid),
        grid=(grid,),
        in_specs=[
            pl.BlockSpec((L, E), lambda i: (i, 0)),
            pl.BlockSpec((L, E), lambda i: (i, 0)),
        ],
        out_specs=pl.BlockSpec(memory_space=pltpu.SMEM),
        out_shape=jax.ShapeDtypeStruct((4,), jnp.float32),
    )(predictions, gate_logits)
    return out[3]


# speedup vs baseline: 1.5741x; 1.5741x over previous
import functools

import jax
import jax.numpy as jnp
from jax.experimental import pallas as pl
from jax.experimental.pallas import tpu as pltpu

_TOP_K = 2
_LAMBDA_RANKING = 0.3
_MARGIN = 0.1
_W_TOP10 = 3.0
_W_TOP11_30 = 1.5
_W_OTHERS = 0.5
_TOP_N = 10
_LANES = 512


def _loss_kernel(p_ref, g_ref, out_ref, *, n_tokens, n_experts, grid):
    i = pl.program_id(0)

    @pl.when(i == 0)
    def _init():
        out_ref[0] = 0.0
        out_ref[1] = 0.0
        out_ref[2] = 0.0
        out_ref[3] = 0.0

    p = p_ref[...]
    g = g_ref[...]
    E = n_experts
    L = p.shape[1]

    b = jax.lax.bitcast_convert_type(g, jnp.int32)
    k = jnp.where(b >= 0, b, b ^ jnp.int32(0x7FFFFFFF))
    aidx = jax.lax.broadcasted_iota(jnp.int32, (E, L), 0)
    c = (k & jnp.int32(~63)) | (jnp.int32(E - 1) - aidx)

    ranks = jnp.zeros((E, L), jnp.int32)
    for bb in range(E):
        cb = c[bb : bb + 1, :]
        ranks = ranks + (cb > c).astype(jnp.int32)

    lt2 = ranks < _TOP_K
    lt10 = ranks < _TOP_N
    lt30 = ranks < 30
    base = jnp.maximum(p, 0.0) + jnp.log(1.0 + jnp.exp(-jnp.abs(p)))
    w = _W_OTHERS + jnp.where(lt30, 1.0, 0.0) + jnp.where(lt10, 1.5, 0.0)
    wsum = w * base - _W_TOP10 * jnp.where(lt2, p, 0.0)
    s_wbce = jnp.sum(wsum)

    qs = [
        jnp.sum(jnp.where(ranks == r, p, 0.0), axis=0, keepdims=True)
        for r in range(_TOP_N)
    ]
    q16 = jnp.concatenate(qs + [jnp.zeros((6, L), jnp.float32)], axis=0)

    ridx = jax.lax.broadcasted_iota(jnp.int32, (16, 1), 0)
    pair = jnp.zeros((16, L), jnp.float32)
    for s in range(1, _TOP_N):
        qshift = jnp.concatenate([q16[s:], q16[:s]], axis=0)
        t = jnp.maximum(_MARGIN - (q16 - qshift), 0.0)
        pair = pair + jnp.where(ridx < _TOP_N - s, t, 0.0)

    out_ref[0] = out_ref[0] + s_wbce
    out_ref[1] = out_ref[1] + jnp.sum(pair)

    @pl.when(i == grid - 1)
    def _final():
        wb = out_ref[0] * (1.0 / float(n_tokens * n_experts))
        n_pairs = _TOP_N * (_TOP_N - 1) // 2
        rank_loss = out_ref[1] * (1.0 / float(n_pairs * n_tokens))
        out_ref[3] = wb + _LAMBDA_RANKING * rank_loss


def kernel(predictions, gate_logits):
    N, E = predictions.shape
    L = min(_LANES, N)
    grid = N // L

    out = pl.pallas_call(
        functools.partial(_loss_kernel, n_tokens=N, n_experts=E, grid=grid),
        grid=(grid,),
        in_specs=[
            pl.BlockSpec((E, L), lambda i: (0, i)),
            pl.BlockSpec((E, L), lambda i: (0, i)),
        ],
        out_specs=pl.BlockSpec(memory_space=pltpu.SMEM),
        out_shape=jax.ShapeDtypeStruct((4,), jnp.float32),
    )(predictions.T, gate_logits.T)
    return out[3]


# speedup vs baseline: 1.5877x; 1.0087x over previous
import functools

import jax
import jax.numpy as jnp
from jax.experimental import pallas as pl
from jax.experimental.pallas import tpu as pltpu

_TOP_K = 2
_LAMBDA_RANKING = 0.3
_MARGIN = 0.1
_W_TOP10 = 3.0
_W_TOP11_30 = 1.5
_W_OTHERS = 0.5
_TOP_N = 10
_LANES = 512


def _loss_kernel(p_ref, g_ref, out_ref, *, n_tokens, n_experts, grid):
    i = pl.program_id(0)

    @pl.when(i == 0)
    def _init():
        out_ref[0] = 0.0
        out_ref[1] = 0.0
        out_ref[2] = 0.0
        out_ref[3] = 0.0

    p = p_ref[...]
    g = g_ref[...]
    E = n_experts
    L = p.shape[1]

    b = jax.lax.bitcast_convert_type(g, jnp.int32)
    k = jnp.where(b >= 0, b, b ^ jnp.int32(0x7FFFFFFF))
    aidx = jax.lax.broadcasted_iota(jnp.int32, (E, L), 0)
    c = (k & jnp.int32(~63)) | (jnp.int32(E - 1) - aidx)

    acc0 = jnp.zeros((E, L), jnp.int32)
    acc1 = jnp.zeros((E, L), jnp.int32)
    for bb in range(0, E, 2):
        acc0 = acc0 + (c[bb : bb + 1, :] > c).astype(jnp.int32)
        acc1 = acc1 + (c[bb + 1 : bb + 2, :] > c).astype(jnp.int32)
    ranks = acc0 + acc1

    lt2 = ranks < _TOP_K
    lt10 = ranks < _TOP_N
    lt30 = ranks < 30
    base = jnp.maximum(p, 0.0) + jnp.log(1.0 + jnp.exp(-jnp.abs(p)))
    w = jnp.where(lt30, jnp.where(lt10, _W_TOP10, _W_TOP11_30), _W_OTHERS)
    wsum = w * base - _W_TOP10 * jnp.where(lt2, p, 0.0)
    s_wbce = jnp.sum(wsum)

    qs = [
        jnp.sum(jnp.where(ranks == r, p, 0.0), axis=0, keepdims=True)
        for r in range(_TOP_N)
    ]
    q16 = jnp.concatenate(qs + [jnp.zeros((6, L), jnp.float32)], axis=0)

    ridx = jax.lax.broadcasted_iota(jnp.int32, (16, 1), 0)
    pair = jnp.zeros((16, L), jnp.float32)
    for s in range(1, _TOP_N):
        qshift = jnp.concatenate([q16[s:], q16[:s]], axis=0)
        t = jnp.maximum(_MARGIN - (q16 - qshift), 0.0)
        pair = pair + jnp.where(ridx < _TOP_N - s, t, 0.0)

    out_ref[0] = out_ref[0] + s_wbce
    out_ref[1] = out_ref[1] + jnp.sum(pair)

    @pl.when(i == grid - 1)
    def _final():
        wb = out_ref[0] * (1.0 / float(n_tokens * n_experts))
        n_pairs = _TOP_N * (_TOP_N - 1) // 2
        rank_loss = out_ref[1] * (1.0 / float(n_pairs * n_tokens))
        out_ref[3] = wb + _LAMBDA_RANKING * rank_loss


def kernel(predictions, gate_logits):
    N, E = predictions.shape
    L = min(_LANES, N)
    grid = N // L

    out = pl.pallas_call(
        functools.partial(_loss_kernel, n_tokens=N, n_experts=E, grid=grid),
        grid=(grid,),
        in_specs=[
            pl.BlockSpec((E, L), lambda i: (0, i)),
            pl.BlockSpec((E, L), lambda i: (0, i)),
        ],
        out_specs=pl.BlockSpec(memory_space=pltpu.SMEM),
        out_shape=jax.ShapeDtypeStruct((4,), jnp.float32),
    )(predictions.T, gate_logits.T)
    return out[3]


# speedup vs baseline: 1.5965x; 1.0056x over previous
import functools

import jax
import jax.numpy as jnp
from jax.experimental import pallas as pl
from jax.experimental.pallas import tpu as pltpu

_TOP_K = 2
_LAMBDA_RANKING = 0.3
_MARGIN = 0.1
_W_TOP10 = 3.0
_W_TOP11_30 = 1.5
_W_OTHERS = 0.5
_TOP_N = 10
_LANES = 1024


def _loss_kernel(p_ref, g_ref, out_ref, *, n_tokens, n_experts, grid):
    i = pl.program_id(0)

    @pl.when(i == 0)
    def _init():
        out_ref[0] = 0.0
        out_ref[1] = 0.0
        out_ref[2] = 0.0
        out_ref[3] = 0.0

    p = p_ref[...]
    g = g_ref[...]
    E = n_experts
    L = p.shape[1]

    b = jax.lax.bitcast_convert_type(g, jnp.int32)
    k = jnp.where(b >= 0, b, b ^ jnp.int32(0x7FFFFFFF))
    aidx = jax.lax.broadcasted_iota(jnp.int32, (E, L), 0)
    c = (k & jnp.int32(~63)) | (jnp.int32(E - 1) - aidx)

    acc0 = jnp.zeros((E, L), jnp.int32)
    acc1 = jnp.zeros((E, L), jnp.int32)
    for bb in range(0, E, 2):
        acc0 = acc0 + (c[bb : bb + 1, :] > c).astype(jnp.int32)
        acc1 = acc1 + (c[bb + 1 : bb + 2, :] > c).astype(jnp.int32)
    ranks = acc0 + acc1

    lt2 = ranks < _TOP_K
    lt10 = ranks < _TOP_N
    lt30 = ranks < 30
    base = jnp.maximum(p, 0.0) + jnp.log(1.0 + jnp.exp(-jnp.abs(p)))
    w = jnp.where(lt30, jnp.where(lt10, _W_TOP10, _W_TOP11_30), _W_OTHERS)
    wsum = w * base - _W_TOP10 * jnp.where(lt2, p, 0.0)
    s_wbce = jnp.sum(wsum)

    qs = [
        jnp.sum(jnp.where(ranks == r, p, 0.0), axis=0, keepdims=True)
        for r in range(_TOP_N)
    ]
    q16 = jnp.concatenate(qs + [jnp.zeros((6, L), jnp.float32)], axis=0)

    ridx = jax.lax.broadcasted_iota(jnp.int32, (16, 1), 0)
    pair = jnp.zeros((16, L), jnp.float32)
    for s in range(1, _TOP_N):
        qshift = jnp.concatenate([q16[s:], q16[:s]], axis=0)
        t = jnp.maximum(_MARGIN - (q16 - qshift), 0.0)
        pair = pair + jnp.where(ridx < _TOP_N - s, t, 0.0)

    out_ref[0] = out_ref[0] + s_wbce
    out_ref[1] = out_ref[1] + jnp.sum(pair)

    @pl.when(i == grid - 1)
    def _final():
        wb = out_ref[0] * (1.0 / float(n_tokens * n_experts))
        n_pairs = _TOP_N * (_TOP_N - 1) // 2
        rank_loss = out_ref[1] * (1.0 / float(n_pairs * n_tokens))
        out_ref[3] = wb + _LAMBDA_RANKING * rank_loss


def kernel(predictions, gate_logits):
    N, E = predictions.shape
    L = min(_LANES, N)
    grid = N // L

    out = pl.pallas_call(
        functools.partial(_loss_kernel, n_tokens=N, n_experts=E, grid=grid),
        grid=(grid,),
        in_specs=[
            pl.BlockSpec((E, L), lambda i: (0, i)),
            pl.BlockSpec((E, L), lambda i: (0, i)),
        ],
        out_specs=pl.BlockSpec(memory_space=pltpu.SMEM),
        out_shape=jax.ShapeDtypeStruct((4,), jnp.float32),
    )(predictions.T, gate_logits.T)
    return out[3]


# speedup vs baseline: 1.6831x; 1.0542x over previous
import functools

import jax
import jax.numpy as jnp
from jax.experimental import pallas as pl
from jax.experimental.pallas import tpu as pltpu

_TOP_K = 2
_LAMBDA_RANKING = 0.3
_MARGIN = 0.1
_W_TOP10 = 3.0
_W_TOP11_30 = 1.5
_W_OTHERS = 0.5
_TOP_N = 10
_LANES = 1024


def _loss_kernel(p_ref, g_ref, out_ref, *, n_tokens, n_experts, grid):
    i = pl.program_id(0)

    @pl.when(i == 0)
    def _init():
        out_ref[0] = 0.0
        out_ref[1] = 0.0
        out_ref[2] = 0.0
        out_ref[3] = 0.0

    E = n_experts
    LB = p_ref.shape[1]
    half = LB // 2
    s0, s1, p0, p1 = 0.0, 0.0, 0.0, 0.0
    for lo in (0, half):
        p = p_ref[:, lo : lo + half]
        g = g_ref[:, lo : lo + half]
        sw, sp = _chunk(p, g, E)
        s0, s1 = s0 + sw, s1 + sp

    out_ref[0] = out_ref[0] + s0
    out_ref[1] = out_ref[1] + s1

    @pl.when(i == grid - 1)
    def _final():
        wb = out_ref[0] * (1.0 / float(n_tokens * n_experts))
        n_pairs = _TOP_N * (_TOP_N - 1) // 2
        rank_loss = out_ref[1] * (1.0 / float(n_pairs * n_tokens))
        out_ref[3] = wb + _LAMBDA_RANKING * rank_loss


def _chunk(p, g, E):
    L = p.shape[1]
    b = jax.lax.bitcast_convert_type(g, jnp.int32)
    k = jnp.where(b >= 0, b, b ^ jnp.int32(0x7FFFFFFF))
    aidx = jax.lax.broadcasted_iota(jnp.int32, (E, L), 0)
    c = (k & jnp.int32(~63)) | (jnp.int32(E - 1) - aidx)

    acc0 = jnp.zeros((E, L), jnp.int32)
    acc1 = jnp.zeros((E, L), jnp.int32)
    for bb in range(0, E, 2):
        acc0 = acc0 + (c[bb : bb + 1, :] > c).astype(jnp.int32)
        acc1 = acc1 + (c[bb + 1 : bb + 2, :] > c).astype(jnp.int32)
    ranks = acc0 + acc1

    lt2 = ranks < _TOP_K
    lt10 = ranks < _TOP_N
    lt30 = ranks < 30
    base = jnp.maximum(p, 0.0) + jnp.log(1.0 + jnp.exp(-jnp.abs(p)))
    w = jnp.where(lt30, jnp.where(lt10, _W_TOP10, _W_TOP11_30), _W_OTHERS)
    wsum = w * base - _W_TOP10 * jnp.where(lt2, p, 0.0)
    s_wbce = jnp.sum(wsum)

    qs = [
        jnp.sum(jnp.where(ranks == r, p, 0.0), axis=0, keepdims=True)
        for r in range(_TOP_N)
    ]
    q16 = jnp.concatenate(qs + [jnp.zeros((6, L), jnp.float32)], axis=0)

    ridx = jax.lax.broadcasted_iota(jnp.int32, (16, 1), 0)
    pair = jnp.zeros((16, L), jnp.float32)
    for s in range(1, _TOP_N):
        qshift = jnp.concatenate([q16[s:], q16[:s]], axis=0)
        t = jnp.maximum(_MARGIN - (q16 - qshift), 0.0)
        pair = pair + jnp.where(ridx < _TOP_N - s, t, 0.0)

    return s_wbce, jnp.sum(pair)


def kernel(predictions, gate_logits):
    N, E = predictions.shape
    L = min(_LANES, N)
    grid = N // L

    out = pl.pallas_call(
        functools.partial(_loss_kernel, n_tokens=N, n_experts=E, grid=grid),
        grid=(grid,),
        in_specs=[
            pl.BlockSpec((E, L), lambda i: (0, i)),
            pl.BlockSpec((E, L), lambda i: (0, i)),
        ],
        out_specs=pl.BlockSpec(memory_space=pltpu.SMEM),
        out_shape=jax.ShapeDtypeStruct((4,), jnp.float32),
    )(predictions.T, gate_logits.T)
    return out[3]


# speedup vs baseline: 1.7494x; 1.0394x over previous
import functools

import jax
import jax.numpy as jnp
from jax.experimental import pallas as pl
from jax.experimental.pallas import tpu as pltpu

_TOP_K = 2
_LAMBDA_RANKING = 0.3
_MARGIN = 0.1
_W_TOP10 = 3.0
_W_TOP11_30 = 1.5
_W_OTHERS = 0.5
_TOP_N = 10
_LANES = 8192


def _loss_kernel(p_ref, g_ref, out_ref, *, n_tokens, n_experts, grid):
    i = pl.program_id(0)

    @pl.when(i == 0)
    def _init():
        out_ref[0] = 0.0
        out_ref[1] = 0.0
        out_ref[2] = 0.0
        out_ref[3] = 0.0

    E = n_experts
    LB = p_ref.shape[1]
    chunk = 512 if LB % 512 == 0 else LB
    s0, s1 = 0.0, 0.0
    for lo in range(0, LB, chunk):
        p = p_ref[:, lo : lo + chunk]
        g = g_ref[:, lo : lo + chunk]
        sw, sp = _chunk(p, g, E)
        s0, s1 = s0 + sw, s1 + sp

    out_ref[0] = out_ref[0] + s0
    out_ref[1] = out_ref[1] + s1

    @pl.when(i == grid - 1)
    def _final():
        wb = out_ref[0] * (1.0 / float(n_tokens * n_experts))
        n_pairs = _TOP_N * (_TOP_N - 1) // 2
        rank_loss = out_ref[1] * (1.0 / float(n_pairs * n_tokens))
        out_ref[3] = wb + _LAMBDA_RANKING * rank_loss


def _chunk(p, g, E):
    L = p.shape[1]
    b = jax.lax.bitcast_convert_type(g, jnp.int32)
    k = jnp.where(b >= 0, b, b ^ jnp.int32(0x7FFFFFFF))
    aidx = jax.lax.broadcasted_iota(jnp.int32, (E, L), 0)
    c = (k & jnp.int32(~63)) | (jnp.int32(E - 1) - aidx)

    acc0 = jnp.zeros((E, L), jnp.int32)
    acc1 = jnp.zeros((E, L), jnp.int32)
    for bb in range(0, E, 2):
        acc0 = acc0 + (c[bb : bb + 1, :] > c).astype(jnp.int32)
        acc1 = acc1 + (c[bb + 1 : bb + 2, :] > c).astype(jnp.int32)
    ranks = acc0 + acc1

    lt2 = ranks < _TOP_K
    lt10 = ranks < _TOP_N
    lt30 = ranks < 30
    base = jnp.maximum(p, 0.0) + jnp.log(1.0 + jnp.exp(-jnp.abs(p)))
    w = jnp.where(lt30, jnp.where(lt10, _W_TOP10, _W_TOP11_30), _W_OTHERS)
    wsum = w * base - _W_TOP10 * jnp.where(lt2, p, 0.0)
    s_wbce = jnp.sum(wsum)

    qs = [
        jnp.sum(jnp.where(ranks == r, p, 0.0), axis=0, keepdims=True)
        for r in range(_TOP_N)
    ]
    q16 = jnp.concatenate(qs + [jnp.zeros((6, L), jnp.float32)], axis=0)

    ridx = jax.lax.broadcasted_iota(jnp.int32, (16, 1), 0)
    pair = jnp.zeros((16, L), jnp.float32)
    for s in range(1, _TOP_N):
        qshift = jnp.concatenate([q16[s:], q16[:s]], axis=0)
        t = jnp.maximum(_MARGIN - (q16 - qshift), 0.0)
        pair = pair + jnp.where(ridx < _TOP_N - s, t, 0.0)

    return s_wbce, jnp.sum(pair)


def kernel(predictions, gate_logits):
    N, E = predictions.shape
    L = min(_LANES, N)
    grid = N // L

    out = pl.pallas_call(
        functools.partial(_loss_kernel, n_tokens=N, n_experts=E, grid=grid),
        grid=(grid,),
        in_specs=[
            pl.BlockSpec((E, L), lambda i: (0, i)),
            pl.BlockSpec((E, L), lambda i: (0, i)),
        ],
        out_specs=pl.BlockSpec(memory_space=pltpu.SMEM),
        out_shape=jax.ShapeDtypeStruct((4,), jnp.float32),
    )(predictions.T, gate_logits.T)
    return out[3]
